# baseline (speedup 1.0000x reference)
import functools

import jax
import jax.numpy as jnp
from jax import lax
from jax.experimental import pallas as pl
from jax.experimental.pallas import tpu as pltpu
from jax.experimental.pallas import tpu_sc as plsc

N = 10000
D = 128
E = 320000
NC, NS, L = 2, 16, 16
NW = NC * NS
NP = 10112
RPT = NP // NS
SINK = 10008
B = 32
CH = 314
CHP = CH + 2
EPT = E // NW

f32 = jnp.float32

_sc_mesh = plsc.VectorSubcoreMesh(
    core_axis_name="c", subcore_axis_name="s", num_cores=NC, num_subcores=NS
)


@functools.partial(
    pl.kernel,
    out_type=jax.ShapeDtypeStruct((NW, NP), f32),
    mesh=_sc_mesh,
    scratch_types=[
        pltpu.VMEM((2, 2, B), jnp.int32),
        pltpu.VMEM((NP,), f32),
        pltpu.SemaphoreType.DMA,
        pltpu.SemaphoreType.DMA,
    ],
    compiler_params=pltpu.CompilerParams(needs_layout_passes=False),
)
def _deg_kernel(sd_hbm, out_hbm, ring, hist, semi0, semi1):
    c = lax.axis_index("c")
    s = lax.axis_index("s")
    w = c * NS + s

    ones16 = jnp.full((L,), 1.0, f32)
    zeros16 = jnp.zeros((L,), f32)

    def zfill(i, carry):
        hist[pl.ds(i * L, L)] = zeros16
        return carry

    lax.fori_loop(0, NP // L, zfill, 0)

    pltpu.async_copy(sd_hbm.at[w, 0], ring.at[0], semi0)
    pltpu.async_copy(sd_hbm.at[w, 1], ring.at[1], semi1)

    def scat(b, j):
        for k in range(B // L):
            idx = ring[b, 1, pl.ds(k * L, L)]
            plsc.addupdate_scatter(hist, [idx], ones16)

    def body(g, carry):
        j = g * 2
        pltpu.make_async_copy(sd_hbm.at[w, j], ring.at[0], semi0).wait()
        scat(0, j)
        pltpu.async_copy(sd_hbm.at[w, j + 2], ring.at[0], semi0)
        pltpu.make_async_copy(sd_hbm.at[w, j + 1], ring.at[1], semi1).wait()
        scat(1, j + 1)
        pltpu.async_copy(sd_hbm.at[w, j + 3], ring.at[1], semi1)
        return carry

    lax.fori_loop(0, CH // 2, body, 0)
    pltpu.make_async_copy(sd_hbm.at[w, CH], ring.at[0], semi0).wait()
    pltpu.make_async_copy(sd_hbm.at[w, CH + 1], ring.at[1], semi1).wait()
    pltpu.sync_copy(hist, out_hbm.at[w])


@functools.partial(
    pl.kernel,
    out_type=jax.ShapeDtypeStruct((NC, NP, D), f32),
    mesh=_sc_mesh,
    scratch_types=[
        pltpu.VMEM((2, 2, B), jnp.int32),
        pltpu.VMEM((B, D), f32),
        pltpu.VMEM((B, D), f32),
        pltpu.VMEM_SHARED((NP, D), f32),
        pltpu.SemaphoreType.DMA,
        pltpu.SemaphoreType.DMA,
        pltpu.SemaphoreType.DMA,
        pltpu.SemaphoreType.DMA,
    ],
)
def _agg_kernel(h_hbm, sd_hbm, out_hbm,
                ring, buf0, buf1, acc, semi0, semi1, semg0, semg1):
    c = lax.axis_index("c")
    s = lax.axis_index("s")
    w = c * NS + s

    zeros16 = jnp.zeros((L,), f32)

    def zrow(i, carry):
        for k in range(D // L):
            buf0[i, pl.ds(k * L, L)] = zeros16
        return carry

    lax.fori_loop(0, B, zrow, 0)

    base = s * RPT

    def zcopy(i, carry):
        pltpu.sync_copy(buf0, acc.at[pl.ds(base + i * B, B)])
        return carry

    nfull = RPT // B
    lax.fori_loop(0, nfull, zcopy, 0)
    rem = RPT - nfull * B
    pltpu.sync_copy(buf0.at[pl.ds(0, rem)], acc.at[pl.ds(base + RPT - rem, rem)])
    plsc.subcore_barrier()

    pltpu.async_copy(sd_hbm.at[w, 0], ring.at[0], semi0)
    pltpu.async_copy(sd_hbm.at[w, 1], ring.at[1], semi1)
    pltpu.make_async_copy(sd_hbm.at[w, 0], ring.at[0], semi0).wait()
    pltpu.async_copy(h_hbm.at[ring.at[0, 0]], buf0, semg0)

    def body(g, carry):
        j = g * 2
        pltpu.make_async_copy(sd_hbm.at[w, j + 1], ring.at[1], semi1).wait()
        pltpu.make_async_copy(h_hbm.at[ring.at[0, 0]], buf0, semg0).wait()
        pltpu.async_copy(h_hbm.at[ring.at[1, 0]], buf1, semg1)
        pltpu.sync_copy(buf0, acc.at[ring.at[0, 1]], add=True)
        pltpu.async_copy(sd_hbm.at[w, j + 2], ring.at[0], semi0)
        pltpu.make_async_copy(h_hbm.at[ring.at[1, 0]], buf1, semg1).wait()
        pltpu.make_async_copy(sd_hbm.at[w, j + 2], ring.at[0], semi0).wait()
        pltpu.async_copy(h_hbm.at[ring.at[0, 0]], buf0, semg0)
        pltpu.sync_copy(buf1, acc.at[ring.at[1, 1]], add=True)
        pltpu.async_copy(sd_hbm.at[w, j + 3], ring.at[1], semi1)
        return carry

    lax.fori_loop(0, CH // 2, body, 0)
    pltpu.make_async_copy(h_hbm.at[ring.at[0, 0]], buf0, semg0).wait()
    pltpu.make_async_copy(sd_hbm.at[w, CH + 1], ring.at[1], semi1).wait()
    plsc.subcore_barrier()
    pltpu.sync_copy(acc.at[pl.ds(base, RPT)], out_hbm.at[c, pl.ds(base, RPT)])


def _tc1_body(x_ref, w_ref, degs_ref, h1p_ref, dinv_ref):
    deg = jnp.sum(degs_ref[...], axis=1, keepdims=True) + 1.0
    dinv = lax.rsqrt(deg)
    dinv_ref[...] = dinv
    h = jnp.dot(x_ref[...], w_ref[...], preferred_element_type=f32)
    h1p_ref[...] = h * dinv[:N]


def _tc2_body(acc_ref, hp_ref, dinv_ref, b_ref, w_ref, out_ref):
    dinv = dinv_ref[...][:N]
    agg = acc_ref[0, :N, :] + acc_ref[1, :N, :] + hp_ref[...]
    x2 = jnp.maximum(agg * dinv + b_ref[...][None, :], 0.0)
    out_ref[...] = jnp.dot(x2, w_ref[...], preferred_element_type=f32) * dinv


def _tc3_body(acc_ref, hp_ref, dinv_ref, b_ref, out_ref):
    dinv = dinv_ref[...][:N]
    agg = acc_ref[0, :N, :] + acc_ref[1, :N, :] + hp_ref[...]
    out_ref[...] = jnp.maximum(agg * dinv + b_ref[...][None, :], 0.0)


_tc1 = pl.pallas_call(
    _tc1_body,
    out_shape=(
        jax.ShapeDtypeStruct((N, D), f32),
        jax.ShapeDtypeStruct((NP, 1), f32),
    ),
)
_tc2 = pl.pallas_call(_tc2_body, out_shape=jax.ShapeDtypeStruct((N, D), f32))
_tc3 = pl.pallas_call(_tc3_body, out_shape=jax.ShapeDtypeStruct((N, D), f32))


def kernel(x, edge_index, batch, W1, b1, W2, b2):
    pad = CHP * B - EPT
    src = edge_index[0].astype(jnp.int32).reshape(NW, EPT)
    dst = edge_index[1].astype(jnp.int32).reshape(NW, EPT)
    src = jnp.concatenate([src, jnp.zeros((NW, pad), jnp.int32)], axis=1)
    dst = jnp.concatenate([dst, jnp.full((NW, pad), SINK, jnp.int32)], axis=1)
    sd = jnp.stack(
        [src.reshape(NW, CHP, B), dst.reshape(NW, CHP, B)], axis=2)

    degs = _deg_kernel(sd)
    h1p, dinv = _tc1(x, W1, degs.T)
    acc1 = _agg_kernel(h1p, sd)
    h2p = _tc2(acc1, h1p, dinv, b1, W2)
    acc2 = _agg_kernel(h2p, sd)
    return _tc3(acc2, h2p, dinv, b2)


# speedup vs baseline: 1.0426x; 1.0426x over previous
import functools

import jax
import jax.numpy as jnp
from jax import lax
from jax.experimental import pallas as pl
from jax.experimental.pallas import tpu as pltpu
from jax.experimental.pallas import tpu_sc as plsc

N = 10000
D = 128
E = 320000
NC, NS, L = 2, 16, 16
NW = NC * NS
NP = 10112
RPT = NP // NS
SINK = 10008
B = 48
CH = 210
CHP = CH + 2
EPT = E // NW

f32 = jnp.float32

_sc_mesh = plsc.VectorSubcoreMesh(
    core_axis_name="c", subcore_axis_name="s", num_cores=NC, num_subcores=NS
)


@functools.partial(
    pl.kernel,
    out_type=jax.ShapeDtypeStruct((NW, NP), f32),
    mesh=_sc_mesh,
    scratch_types=[
        pltpu.VMEM((2, 2, B), jnp.int32),
        pltpu.VMEM((NP,), f32),
        pltpu.SemaphoreType.DMA,
        pltpu.SemaphoreType.DMA,
    ],
    compiler_params=pltpu.CompilerParams(needs_layout_passes=False),
)
def _deg_kernel(sd_hbm, out_hbm, ring, hist, semi0, semi1):
    c = lax.axis_index("c")
    s = lax.axis_index("s")
    w = c * NS + s

    ones16 = jnp.full((L,), 1.0, f32)
    zeros16 = jnp.zeros((L,), f32)

    def zfill(i, carry):
        hist[pl.ds(i * L, L)] = zeros16
        return carry

    lax.fori_loop(0, NP // L, zfill, 0)

    pltpu.async_copy(sd_hbm.at[w, 0], ring.at[0], semi0)
    pltpu.async_copy(sd_hbm.at[w, 1], ring.at[1], semi1)

    def scat(b, j):
        for k in range(B // L):
            idx = ring[b, 1, pl.ds(k * L, L)]
            plsc.addupdate_scatter(hist, [idx], ones16)

    def body(g, carry):
        j = g * 2
        pltpu.make_async_copy(sd_hbm.at[w, j], ring.at[0], semi0).wait()
        scat(0, j)
        pltpu.async_copy(sd_hbm.at[w, j + 2], ring.at[0], semi0)
        pltpu.make_async_copy(sd_hbm.at[w, j + 1], ring.at[1], semi1).wait()
        scat(1, j + 1)
        pltpu.async_copy(sd_hbm.at[w, j + 3], ring.at[1], semi1)
        return carry

    lax.fori_loop(0, CH // 2, body, 0)
    pltpu.make_async_copy(sd_hbm.at[w, CH], ring.at[0], semi0).wait()
    pltpu.make_async_copy(sd_hbm.at[w, CH + 1], ring.at[1], semi1).wait()
    pltpu.sync_copy(hist, out_hbm.at[w])


@functools.partial(
    pl.kernel,
    out_type=jax.ShapeDtypeStruct((NC, NP, D), f32),
    mesh=_sc_mesh,
    scratch_types=[
        pltpu.VMEM((2, 2, B), jnp.int32),
        pltpu.VMEM((B, D), f32),
        pltpu.VMEM((B, D), f32),
        pltpu.VMEM_SHARED((NP, D), f32),
        pltpu.SemaphoreType.DMA,
        pltpu.SemaphoreType.DMA,
        pltpu.SemaphoreType.DMA,
        pltpu.SemaphoreType.DMA,
    ],
)
def _agg_kernel(h_hbm, sd_hbm, out_hbm,
                ring, buf0, buf1, acc, semi0, semi1, semg0, semg1):
    c = lax.axis_index("c")
    s = lax.axis_index("s")
    w = c * NS + s

    zeros16 = jnp.zeros((L,), f32)

    def zrow(i, carry):
        for k in range(D // L):
            buf0[i, pl.ds(k * L, L)] = zeros16
        return carry

    lax.fori_loop(0, B, zrow, 0)

    base = s * RPT

    def zcopy(i, carry):
        pltpu.sync_copy(buf0, acc.at[pl.ds(base + i * B, B)])
        return carry

    nfull = RPT // B
    lax.fori_loop(0, nfull, zcopy, 0)
    rem = RPT - nfull * B
    pltpu.sync_copy(buf0.at[pl.ds(0, rem)], acc.at[pl.ds(base + RPT - rem, rem)])
    plsc.subcore_barrier()

    pltpu.async_copy(sd_hbm.at[w, 0], ring.at[0], semi0)
    pltpu.async_copy(sd_hbm.at[w, 1], ring.at[1], semi1)
    pltpu.make_async_copy(sd_hbm.at[w, 0], ring.at[0], semi0).wait()
    pltpu.async_copy(h_hbm.at[ring.at[0, 0]], buf0, semg0)

    def body(g, carry):
        j = g * 2
        pltpu.make_async_copy(sd_hbm.at[w, j + 1], ring.at[1], semi1).wait()
        pltpu.make_async_copy(h_hbm.at[ring.at[0, 0]], buf0, semg0).wait()
        pltpu.async_copy(h_hbm.at[ring.at[1, 0]], buf1, semg1)
        pltpu.sync_copy(buf0, acc.at[ring.at[0, 1]], add=True)
        pltpu.async_copy(sd_hbm.at[w, j + 2], ring.at[0], semi0)
        pltpu.make_async_copy(h_hbm.at[ring.at[1, 0]], buf1, semg1).wait()
        pltpu.make_async_copy(sd_hbm.at[w, j + 2], ring.at[0], semi0).wait()
        pltpu.async_copy(h_hbm.at[ring.at[0, 0]], buf0, semg0)
        pltpu.sync_copy(buf1, acc.at[ring.at[1, 1]], add=True)
        pltpu.async_copy(sd_hbm.at[w, j + 3], ring.at[1], semi1)
        return carry

    lax.fori_loop(0, CH // 2, body, 0)
    pltpu.make_async_copy(h_hbm.at[ring.at[0, 0]], buf0, semg0).wait()
    pltpu.make_async_copy(sd_hbm.at[w, CH + 1], ring.at[1], semi1).wait()
    plsc.subcore_barrier()
    pltpu.sync_copy(acc.at[pl.ds(base, RPT)], out_hbm.at[c, pl.ds(base, RPT)])


def _tc1_body(x_ref, w_ref, degs_ref, h1p_ref, dinv_ref):
    deg = jnp.sum(degs_ref[...], axis=1, keepdims=True) + 1.0
    dinv = lax.rsqrt(deg)
    dinv_ref[...] = dinv
    h = jnp.dot(x_ref[...], w_ref[...], preferred_element_type=f32)
    h1p_ref[...] = h * dinv[:N]


def _tc2_body(acc_ref, hp_ref, dinv_ref, b_ref, w_ref, out_ref):
    dinv = dinv_ref[...][:N]
    agg = acc_ref[0, :N, :] + acc_ref[1, :N, :] + hp_ref[...]
    x2 = jnp.maximum(agg * dinv + b_ref[...][None, :], 0.0)
    out_ref[...] = jnp.dot(x2, w_ref[...], preferred_element_type=f32) * dinv


def _tc3_body(acc_ref, hp_ref, dinv_ref, b_ref, out_ref):
    dinv = dinv_ref[...][:N]
    agg = acc_ref[0, :N, :] + acc_ref[1, :N, :] + hp_ref[...]
    out_ref[...] = jnp.maximum(agg * dinv + b_ref[...][None, :], 0.0)


_tc1 = pl.pallas_call(
    _tc1_body,
    out_shape=(
        jax.ShapeDtypeStruct((N, D), f32),
        jax.ShapeDtypeStruct((NP, 1), f32),
    ),
)
_tc2 = pl.pallas_call(_tc2_body, out_shape=jax.ShapeDtypeStruct((N, D), f32))
_tc3 = pl.pallas_call(_tc3_body, out_shape=jax.ShapeDtypeStruct((N, D), f32))


def kernel(x, edge_index, batch, W1, b1, W2, b2):
    pad = CHP * B - EPT
    src = edge_index[0].astype(jnp.int32).reshape(NW, EPT)
    dst = edge_index[1].astype(jnp.int32).reshape(NW, EPT)
    src = jnp.concatenate([src, jnp.zeros((NW, pad), jnp.int32)], axis=1)
    dst = jnp.concatenate([dst, jnp.full((NW, pad), SINK, jnp.int32)], axis=1)
    sd = jnp.stack(
        [src.reshape(NW, CHP, B), dst.reshape(NW, CHP, B)], axis=2)

    degs = _deg_kernel(sd)
    h1p, dinv = _tc1(x, W1, degs.T)
    acc1 = _agg_kernel(h1p, sd)
    h2p = _tc2(acc1, h1p, dinv, b1, W2)
    acc2 = _agg_kernel(h2p, sd)
    return _tc3(acc2, h2p, dinv, b2)
